# baseline (speedup 1.0000x reference)
import functools

import jax
import jax.numpy as jnp
from jax import lax
from jax.experimental import pallas as pl
from jax.experimental.pallas import tpu as pltpu
from jax.experimental.pallas import tpu_sc as plsc

N_NODES = 10000
N_EDGES = 320000
D_FEAT = 128
D_HID = 128
N_CLS = 64
K_LAYERS = 8
ALPHA = 1.0
LAMBD = 1.0
COEF = 1.0 / (1.0 + ALPHA * LAMBD)
K1 = COEF * ALPHA * LAMBD
K2 = COEF

NC = 2
NS = 16
NW = NC * NS
HC = N_CLS // NC
NPAD = 10240
NPAD_EXTRA = NPAD - N_NODES
SLICE = NPAD // NS
CHUNK = 128
TOT_E = 327680
EPT = TOT_E // NS
CPT = EPT // CHUNK
DEG_CPT = TOT_E // NW // CHUNK
NBUF = 4

_mesh = plsc.VectorSubcoreMesh(core_axis_name="c", subcore_axis_name="s",
                               num_cores=NC, num_subcores=NS)
_sc_params = pltpu.CompilerParams(use_tc_tiling_on_sc=False)


def _deg_body(src_hbm, dst_hbm, ones_hbm, zeros_hbm, deg_hbm,
              srcv, dstv, oneslo, oneshi, deg_sh, dsem):
    c = lax.axis_index("c")
    s = lax.axis_index("s")
    w = c * NS + s
    pltpu.sync_copy(src_hbm.at[pl.ds(w * DEG_CPT, DEG_CPT)], srcv)
    pltpu.sync_copy(dst_hbm.at[pl.ds(w * DEG_CPT, DEG_CPT)], dstv)
    pltpu.sync_copy(ones_hbm.at[0], oneslo)
    pltpu.sync_copy(ones_hbm.at[1], oneshi)
    pltpu.sync_copy(zeros_hbm.at[pl.ds(s * SLICE, SLICE)],
                    deg_sh.at[pl.ds(s * SLICE, SLICE)])
    plsc.subcore_barrier()

    def chunk(j, carry):
        pltpu.async_copy(oneslo, deg_sh.at[srcv.at[j]], dsem, add=True)
        pltpu.async_copy(oneshi, deg_sh.at[dstv.at[j]], dsem, add=True)
        return carry

    lax.fori_loop(0, DEG_CPT, chunk, 0)

    def drain(j, carry):
        pltpu.make_async_copy(ones_hbm.at[0], oneslo, dsem).wait()
        return carry

    lax.fori_loop(0, 2 * DEG_CPT, drain, 0)
    plsc.subcore_barrier()
    pltpu.sync_copy(deg_sh.at[pl.ds(s * SLICE, SLICE)],
                    deg_hbm.at[c, pl.ds(s * SLICE, SLICE)])


_deg_call = pl.kernel(
    _deg_body,
    out_type=jax.ShapeDtypeStruct((NC, NPAD, 16), jnp.float32),
    mesh=_mesh,
    scratch_types=[
        pltpu.VMEM((DEG_CPT, CHUNK), jnp.int32),
        pltpu.VMEM((DEG_CPT, CHUNK), jnp.int32),
        pltpu.VMEM((CHUNK, 16), jnp.float32),
        pltpu.VMEM((CHUNK, 16), jnp.float32),
        pltpu.VMEM_SHARED((NPAD, 16), jnp.float32),
        pltpu.SemaphoreType.DMA,
    ],
    compiler_params=_sc_params,
)


def _prop_body(u0_hbm, c_hbm, pk_hbm, zeros_hbm, u8_hbm,
               pkv, sring, dring, rowsv, u0v, cv, zerov, u_sh, agg_sh,
               gsem_a, gsem_b, ssem_a, ssem_b):
    c = lax.axis_index("c")
    s = lax.axis_index("s")
    nsl = pl.ds(s * SLICE, SLICE)
    pltpu.sync_copy(pk_hbm.at[pl.ds(s * CPT, CPT)], pkv)
    pltpu.sync_copy(u0_hbm.at[c, nsl], u0v)
    pltpu.sync_copy(u0_hbm.at[c, nsl], u_sh.at[nsl])
    pltpu.sync_copy(c_hbm.at[nsl], cv)
    pltpu.sync_copy(zeros_hbm, zerov)

    def zero_agg():
        for z in range(SLICE // CHUNK):
            pltpu.sync_copy(zerov,
                            agg_sh.at[pl.ds(s * SLICE + z * CHUNK, CHUNK)])

    zero_agg()
    plsc.subcore_barrier()

    def fire_g(g, bank, sem):
        for t in range(NBUF):
            j = g * NBUF + t
            for v in range(CHUNK // 16):
                slc = pl.ds(v * 16, 16)
                pk = pkv[j, slc]
                sring[bank, t, slc] = lax.shift_right_logical(pk, 14)
                dring[bank, t, slc] = lax.bitwise_and(pk, 16383)
            pltpu.async_copy(u_sh.at[sring.at[bank, t]],
                             rowsv.at[bank, t], sem)

    def fire_s(g, bank, sem):
        for t in range(NBUF):
            pltpu.async_copy(rowsv.at[bank, t],
                             agg_sh.at[dring.at[bank, t]], sem, add=True)

    def wait_n(sem):
        for _ in range(NBUF):
            pltpu.make_async_copy(u0_hbm.at[0, pl.ds(0, CHUNK)],
                                  rowsv.at[0, 0], sem).wait()

    ngroups = CPT // NBUF

    def one_iter(k, carry):
        fire_g(0, 0, gsem_a)
        fire_g(1, 1, gsem_b)
        wait_n(gsem_a)
        fire_s(0, 0, ssem_a)

        def body(ii, cc):
            g0 = ii * 2
            wait_n(ssem_a)
            fire_g(g0, 0, gsem_a)
            wait_n(gsem_b)
            fire_s(g0 - 1, 1, ssem_b)
            wait_n(ssem_b)
            fire_g(g0 + 1, 1, gsem_b)
            wait_n(gsem_a)
            fire_s(g0, 0, ssem_a)
            return cc

        lax.fori_loop(1, ngroups // 2, body, 0)
        wait_n(ssem_a)
        wait_n(gsem_b)
        fire_s(ngroups - 1, 1, ssem_b)
        wait_n(ssem_b)
        plsc.subcore_barrier()

        pieces = [(p // NBUF, p % NBUF) for p in range(SLICE // CHUNK)]
        for p, (pb, pt) in enumerate(pieces):
            pltpu.async_copy(agg_sh.at[pl.ds(s * SLICE + p * CHUNK, CHUNK)],
                             rowsv.at[pb, pt], gsem_a)
        for _ in pieces:
            pltpu.make_async_copy(u0_hbm.at[0, pl.ds(0, CHUNK)],
                                  rowsv.at[0, 0], gsem_a).wait()
        for p in range(len(pieces)):
            pltpu.async_copy(zerov,
                             agg_sh.at[pl.ds(s * SLICE + p * CHUNK, CHUNK)],
                             ssem_a)

        for p, (pb, pt) in enumerate(pieces):
            def upd(n16, cc, p=p, pb=pb, pt=pt):
                cvec = cv[pl.ds(p * CHUNK + n16 * 16, 16)] * K1
                for j in range(16):
                    n = n16 * 16 + j
                    cn = cvec[j]
                    for h in range(HC // 16):
                        slc = pl.ds(h * 16, 16)
                        rowsv[pb, pt, n, slc] = (cn * rowsv[pb, pt, n, slc]
                                                 + K2 * u0v[p * CHUNK + n, slc])
                return cc

            lax.fori_loop(0, CHUNK // 16, upd, 0)
            pltpu.async_copy(rowsv.at[pb, pt],
                             u_sh.at[pl.ds(s * SLICE + p * CHUNK, CHUNK)],
                             ssem_b)
        for _ in pieces:
            pltpu.make_async_copy(u0_hbm.at[0, pl.ds(0, CHUNK)],
                                  rowsv.at[0, 0], ssem_a).wait()
            pltpu.make_async_copy(u0_hbm.at[0, pl.ds(0, CHUNK)],
                                  rowsv.at[0, 0], ssem_b).wait()
        plsc.subcore_barrier()
        return carry

    lax.fori_loop(0, K_LAYERS, one_iter, 0)
    pltpu.sync_copy(u_sh.at[nsl], u8_hbm.at[c, nsl])


_prop_call = pl.kernel(
    _prop_body,
    out_type=jax.ShapeDtypeStruct((NC, NPAD, HC), jnp.float32),
    mesh=_mesh,
    scratch_types=[
        pltpu.VMEM((CPT, CHUNK), jnp.int32),
        pltpu.VMEM((2, NBUF, CHUNK), jnp.int32),
        pltpu.VMEM((2, NBUF, CHUNK), jnp.int32),
        pltpu.VMEM((2, NBUF, CHUNK, HC), jnp.float32),
        pltpu.VMEM((SLICE, HC), jnp.float32),
        pltpu.VMEM((SLICE,), jnp.float32),
        pltpu.VMEM((CHUNK, HC), jnp.float32),
        pltpu.VMEM_SHARED((NPAD, HC), jnp.float32),
        pltpu.VMEM_SHARED((NPAD, HC), jnp.float32),
        pltpu.SemaphoreType.DMA,
        pltpu.SemaphoreType.DMA,
        pltpu.SemaphoreType.DMA,
        pltpu.SemaphoreType.DMA,
    ],
    compiler_params=_sc_params,
)


_RB = 1000
_RBP = 1024


def _mm_body(x_ref, w0_ref, b0_ref, w1_ref, g0_ref):
    h0 = jnp.dot(x_ref[...], w0_ref[...],
                 preferred_element_type=jnp.float32) + b0_ref[...]
    g0_ref[...] = jnp.dot(h0, w1_ref[...], preferred_element_type=jnp.float32)


_mm_call = pl.pallas_call(
    _mm_body,
    grid=(NPAD // _RBP,),
    in_specs=[
        pl.BlockSpec((_RBP, D_FEAT), lambda i: (i, 0)),
        pl.BlockSpec((D_FEAT, D_HID), lambda i: (0, 0)),
        pl.BlockSpec((1, D_HID), lambda i: (0, 0)),
        pl.BlockSpec((D_HID, N_CLS), lambda i: (0, 0)),
    ],
    out_specs=pl.BlockSpec((_RBP, N_CLS), lambda i: (i, 0)),
    out_shape=jax.ShapeDtypeStruct((NPAD, N_CLS), jnp.float32),
)


def _prep_body(g0_ref, deg_ref, u0_ref, c_ref, ia_ref):
    i = pl.program_id(0)
    g0 = g0_ref[...]
    dgo = jnp.maximum(deg_ref[0][:, 0:1] + deg_ref[1][:, 0:1], 1.0)
    dgi = jnp.maximum(deg_ref[0][:, 8:9] + deg_ref[1][:, 8:9], 1.0)
    a = lax.rsqrt(dgo)
    b = lax.rsqrt(dgi)
    c_ref[...] = a * b
    ia_ref[...] = jnp.sqrt(dgo)
    row = i * _RBP + lax.broadcasted_iota(jnp.int32, (_RBP, 1), 0)
    u0 = jnp.where(row < N_NODES, a * g0, 0.0)
    u0_ref[0] = u0[:, :HC]
    u0_ref[1] = u0[:, HC:]


_prep_call = pl.pallas_call(
    _prep_body,
    grid=(NPAD // _RBP,),
    in_specs=[
        pl.BlockSpec((_RBP, N_CLS), lambda i: (i, 0)),
        pl.BlockSpec((NC, _RBP, 16), lambda i: (0, i, 0)),
    ],
    out_specs=(
        pl.BlockSpec((NC, _RBP, HC), lambda i: (0, i, 0)),
        pl.BlockSpec((_RBP, 1), lambda i: (i, 0)),
        pl.BlockSpec((_RBP, 1), lambda i: (i, 0)),
    ),
    out_shape=(
        jax.ShapeDtypeStruct((NC, NPAD, HC), jnp.float32),
        jax.ShapeDtypeStruct((NPAD, 1), jnp.float32),
        jax.ShapeDtypeStruct((NPAD, 1), jnp.float32),
    ),
)


def _fin_body(u8_ref, ia_ref, b1_ref, w2_ref, b2_ref, o_ref):
    g8 = ia_ref[...] * jnp.concatenate([u8_ref[0], u8_ref[1]], axis=1)
    z = jnp.maximum(g8 + b1_ref[...], 0.0)
    o_ref[...] = jnp.dot(z, w2_ref[...],
                         preferred_element_type=jnp.float32) + b2_ref[...]


_fin_call = pl.pallas_call(
    _fin_body,
    grid=(N_NODES // _RB,),
    in_specs=[
        pl.BlockSpec((NC, _RB, HC), lambda i: (0, i, 0)),
        pl.BlockSpec((_RB, 1), lambda i: (i, 0)),
        pl.BlockSpec((1, N_CLS), lambda i: (0, 0)),
        pl.BlockSpec((N_CLS, N_CLS), lambda i: (0, 0)),
        pl.BlockSpec((1, N_CLS), lambda i: (0, 0)),
    ],
    out_specs=pl.BlockSpec((_RB, N_CLS), lambda i: (i, 0)),
    out_shape=jax.ShapeDtypeStruct((N_NODES, N_CLS), jnp.float32),
)


def kernel(features, edge_index, W0, b0, W1, b1, W2, b2):
    src = jnp.asarray(edge_index[0], jnp.int32)
    dst = jnp.asarray(edge_index[1], jnp.int32)
    npd = TOT_E - N_EDGES
    pad = N_NODES + (jnp.arange(npd, dtype=jnp.int32) % NPAD_EXTRA)
    src2 = jnp.concatenate([src, pad]).reshape(TOT_E // CHUNK, CHUNK)
    dst2 = jnp.concatenate([dst, pad]).reshape(TOT_E // CHUNK, CHUNK)

    colid = jnp.arange(16, dtype=jnp.int32)
    ones2 = jnp.stack([jnp.tile((colid < 8).astype(jnp.float32), (CHUNK, 1)),
                       jnp.tile((colid >= 8).astype(jnp.float32), (CHUNK, 1))])
    zeros16 = jnp.zeros((NPAD, 16), jnp.float32)
    zerosHC = jnp.zeros((CHUNK, HC), jnp.float32)

    deg_p = _deg_call(src2, dst2, ones2, zeros16)
    g0 = _mm_call(features, W0, b0.reshape(1, -1), W1)
    u0p, cvec, inva = _prep_call(g0, deg_p)
    cp = cvec.reshape(NPAD)

    pk2 = src2 * 16384 + dst2
    u8 = _prop_call(u0p, cp, pk2, zerosHC)
    return _fin_call(u8, inva, b1.reshape(1, -1), W2, b2.reshape(1, -1))
